# Initial kernel scaffold; baseline (speedup 1.0000x reference)
import jax
import jax.numpy as jnp
from jax.experimental import pallas as pl


def kernel(x, edge_index, W_in, b_in, W_out, b_out, W_fc, b_fc):
    raise NotImplementedError("write your pallas kernel here")


# speedup vs baseline: 11.6165x; 11.6165x over previous
import functools

import jax
import jax.numpy as jnp
from jax import lax
from jax.experimental import pallas as pl
from jax.experimental.pallas import tpu as pltpu
from jax.experimental.pallas import tpu_sc as plsc

N = 10000
E = 160000
D = 256
NPAD = 10240
RPT = NPAD // 16
EPT = E // 16
CHUNK = 128
NFULL = EPT // CHUNK
TAIL = EPT - NFULL * CHUNK
BR = 1024

_MESH = plsc.VectorSubcoreMesh(core_axis_name="c", subcore_axis_name="s")


@functools.partial(
    pl.kernel,
    out_type=(jax.ShapeDtypeStruct((NPAD,), jnp.int32),
              jax.ShapeDtypeStruct((NPAD,), jnp.int32)),
    mesh=_MESH,
    scratch_types=[
        pltpu.VMEM((CHUNK,), jnp.int32),
        pltpu.VMEM((TAIL,), jnp.int32),
        pltpu.VMEM((CHUNK,), jnp.int32),
        pltpu.VMEM((RPT,), jnp.int32),
        pltpu.VMEM_SHARED((NPAD,), jnp.int32),
    ],
)
def _sc_degrees(erow_ref, ecol_ref, cr_ref, cc_ref,
                idx_v, idxt_v, ones_v, row_v, hist_s):
    cid = lax.axis_index("c")
    sid = lax.axis_index("s")

    def phase(tab_ref, out_ref):
        for j in range(RPT // 16):
            row_v[pl.ds(j * 16, 16)] = jnp.zeros((16,), jnp.int32)
        pltpu.sync_copy(row_v, hist_s.at[pl.ds(sid * RPT, RPT)])
        for j in range(CHUNK // 16):
            ones_v[pl.ds(j * 16, 16)] = jnp.ones((16,), jnp.int32)
        plsc.subcore_barrier()
        base = sid * EPT

        def step(i, carry):
            off = base + i * CHUNK
            pltpu.sync_copy(tab_ref.at[pl.ds(off, CHUNK)], idx_v)
            pltpu.sync_copy(ones_v, hist_s.at[idx_v], add=True)
            return carry

        lax.fori_loop(0, NFULL, step, 0)
        toff = base + NFULL * CHUNK
        pltpu.sync_copy(tab_ref.at[pl.ds(toff, TAIL)], idxt_v)
        pltpu.sync_copy(ones_v.at[pl.ds(0, TAIL)], hist_s.at[idxt_v], add=True)
        plsc.subcore_barrier()
        pltpu.sync_copy(hist_s.at[pl.ds(sid * RPT, RPT)], row_v)
        pltpu.sync_copy(row_v, out_ref.at[pl.ds(sid * RPT, RPT)])

    @pl.when(cid == 0)
    def _():
        phase(erow_ref, cr_ref)

    @pl.when(cid == 1)
    def _():
        phase(ecol_ref, cc_ref)


@functools.partial(
    pl.kernel,
    out_type=tuple(jax.ShapeDtypeStruct((NPAD, 128), jnp.float32)
                   for _ in range(4)),
    mesh=_MESH,
    scratch_types=[
        pltpu.VMEM((CHUNK,), jnp.int32),
        pltpu.VMEM((CHUNK,), jnp.int32),
        pltpu.VMEM((TAIL,), jnp.int32),
        pltpu.VMEM((TAIL,), jnp.int32),
        pltpu.VMEM((CHUNK, 128), jnp.float32),
        pltpu.VMEM((TAIL, 128), jnp.float32),
        pltpu.VMEM((CHUNK, 128), jnp.float32),
        pltpu.VMEM_SHARED((NPAD, 128), jnp.float32),
        pltpu.SemaphoreType.DMA,
    ],
)
def _sc_scatter(erow_ref, ecol_ref, mil_ref, mir_ref, mol_ref, mor_ref,
                pil_ref, pir_ref, pol_ref, por_ref,
                si_v, di_v, sit_v, dit_v, g_v, gt_v, row_v, acc_s, sem):
    cid = lax.axis_index("c")
    sid = lax.axis_index("s")

    def do_phase(m_ref, p_ref, s_ref, d_ref):
        for k in range(RPT // CHUNK):
            off = sid * RPT + k * CHUNK
            pltpu.sync_copy(m_ref.at[pl.ds(off, CHUNK)], row_v)
            pltpu.sync_copy(row_v, acc_s.at[pl.ds(off, CHUNK)])
        plsc.subcore_barrier()
        base = sid * EPT

        def step(i, carry):
            off = base + i * CHUNK
            pltpu.sync_copy(s_ref.at[pl.ds(off, CHUNK)], si_v)
            pltpu.sync_copy(d_ref.at[pl.ds(off, CHUNK)], di_v)
            pltpu.async_copy(m_ref.at[si_v], g_v, sem).wait()
            pltpu.sync_copy(g_v, acc_s.at[di_v], add=True)
            return carry

        lax.fori_loop(0, NFULL, step, 0)
        toff = base + NFULL * CHUNK
        pltpu.sync_copy(s_ref.at[pl.ds(toff, TAIL)], sit_v)
        pltpu.sync_copy(d_ref.at[pl.ds(toff, TAIL)], dit_v)
        pltpu.async_copy(m_ref.at[sit_v], gt_v, sem).wait()
        pltpu.sync_copy(gt_v, acc_s.at[dit_v], add=True)
        plsc.subcore_barrier()
        for k in range(RPT // CHUNK):
            off = sid * RPT + k * CHUNK
            pltpu.sync_copy(acc_s.at[pl.ds(off, CHUNK)], row_v)
            pltpu.sync_copy(row_v, p_ref.at[pl.ds(off, CHUNK)])
        plsc.subcore_barrier()

    @pl.when(cid == 0)
    def _():
        do_phase(mil_ref, pil_ref, erow_ref, ecol_ref)
        do_phase(mol_ref, pol_ref, ecol_ref, erow_ref)

    @pl.when(cid == 1)
    def _():
        do_phase(mir_ref, pir_ref, erow_ref, ecol_ref)
        do_phase(mor_ref, por_ref, ecol_ref, erow_ref)


def _tc_prep_body(x_ref, w_ref, dc_ref, dr_ref, mil, mir, mol, mor, di, do_):
    h = jnp.dot(x_ref[...], w_ref[...], preferred_element_type=jnp.float32)
    din = lax.rsqrt(dc_ref[...].astype(jnp.float32) + 1.0)
    dou = lax.rsqrt(dr_ref[...].astype(jnp.float32) + 1.0)
    mil[...] = h[:, 0:128] * din
    mir[...] = h[:, 128:256] * din
    mol[...] = h[:, 256:384] * dou
    mor[...] = h[:, 384:512] * dou
    di[...] = din
    do_[...] = dou


_tc_prep = pl.pallas_call(
    _tc_prep_body,
    grid=(NPAD // BR,),
    in_specs=[
        pl.BlockSpec((BR, D), lambda i: (i, 0)),
        pl.BlockSpec((D, 2 * D), lambda i: (0, 0)),
        pl.BlockSpec((BR, 1), lambda i: (i, 0)),
        pl.BlockSpec((BR, 1), lambda i: (i, 0)),
    ],
    out_specs=[
        pl.BlockSpec((BR, 128), lambda i: (i, 0)),
        pl.BlockSpec((BR, 128), lambda i: (i, 0)),
        pl.BlockSpec((BR, 128), lambda i: (i, 0)),
        pl.BlockSpec((BR, 128), lambda i: (i, 0)),
        pl.BlockSpec((BR, 1), lambda i: (i, 0)),
        pl.BlockSpec((BR, 1), lambda i: (i, 0)),
    ],
    out_shape=[jax.ShapeDtypeStruct((NPAD, 128), jnp.float32)] * 4
    + [jax.ShapeDtypeStruct((NPAD, 1), jnp.float32)] * 2,
)


def _tc_final_body(pil, pir, pol, por, di, do_, wfc, bi, bo, bf, out):
    pin = jnp.concatenate([pil[...], pir[...]], axis=1) * di[...]
    pou = jnp.concatenate([pol[...], por[...]], axis=1) * do_[...]
    big = jnp.concatenate([pin, pou], axis=1)
    y = jnp.dot(big, wfc[...], preferred_element_type=jnp.float32)
    bias = (jnp.dot(bi[...], wfc[0:D, :], preferred_element_type=jnp.float32)
            + jnp.dot(bo[...], wfc[D:2 * D, :],
                      preferred_element_type=jnp.float32)
            + bf[...])
    out[...] = jnp.maximum(y + bias, 0.0)


_tc_final = pl.pallas_call(
    _tc_final_body,
    grid=(NPAD // BR,),
    in_specs=[
        pl.BlockSpec((BR, 128), lambda i: (i, 0)),
        pl.BlockSpec((BR, 128), lambda i: (i, 0)),
        pl.BlockSpec((BR, 128), lambda i: (i, 0)),
        pl.BlockSpec((BR, 128), lambda i: (i, 0)),
        pl.BlockSpec((BR, 1), lambda i: (i, 0)),
        pl.BlockSpec((BR, 1), lambda i: (i, 0)),
        pl.BlockSpec((2 * D, D), lambda i: (0, 0)),
        pl.BlockSpec((1, D), lambda i: (0, 0)),
        pl.BlockSpec((1, D), lambda i: (0, 0)),
        pl.BlockSpec((1, D), lambda i: (0, 0)),
    ],
    out_specs=pl.BlockSpec((BR, D), lambda i: (i, 0)),
    out_shape=jax.ShapeDtypeStruct((NPAD, D), jnp.float32),
)


def kernel(x, edge_index, W_in, b_in, W_out, b_out, W_fc, b_fc):
    ei = edge_index.astype(jnp.int32)
    erow, ecol = ei[0], ei[1]
    xp = jnp.pad(x, ((0, NPAD - N), (0, 0)))
    Wcat = jnp.concatenate([W_in, W_out], axis=1)
    cnt_row, cnt_col = _sc_degrees(erow, ecol)
    mil, mir, mol, mor, din, dou = _tc_prep(
        xp, Wcat, cnt_col[:, None], cnt_row[:, None])
    pil, pir, pol, por = _sc_scatter(erow, ecol, mil, mir, mol, mor)
    y = _tc_final(pil, pir, pol, por, din, dou, W_fc,
                  b_in[None, :], b_out[None, :], b_fc[None, :])
    return y[:N]


# speedup vs baseline: 19.1103x; 1.6451x over previous
import functools

import jax
import jax.numpy as jnp
from jax import lax
from jax.experimental import pallas as pl
from jax.experimental.pallas import tpu as pltpu
from jax.experimental.pallas import tpu_sc as plsc

N = 10000
E = 160000
D = 256
NPAD = 10240
RPT = NPAD // 16
CHUNK = 64
EPAD = 163840
NCH = EPAD // CHUNK
CPT = NCH // 16
NBUF = 2
BR = 1024

_MESH = plsc.VectorSubcoreMesh(core_axis_name="c", subcore_axis_name="s")


@functools.partial(
    pl.kernel,
    out_type=(jax.ShapeDtypeStruct((NPAD,), jnp.int32),
              jax.ShapeDtypeStruct((NPAD,), jnp.int32)),
    mesh=_MESH,
    scratch_types=[
        pltpu.VMEM((CPT, CHUNK), jnp.int32),
        pltpu.VMEM((CHUNK,), jnp.int32),
        pltpu.VMEM((CHUNK,), jnp.int32),
        pltpu.VMEM((RPT,), jnp.int32),
        pltpu.VMEM_SHARED((NPAD,), jnp.int32),
    ],
)
def _sc_degrees(epk_ref, cr_ref, cc_ref, pk_v, uidx_v, ones_v, row_v, hist_s):
    cid = lax.axis_index("c")
    sid = lax.axis_index("s")

    def phase(high, out_ref):
        for j in range(RPT // 16):
            row_v[pl.ds(j * 16, 16)] = jnp.zeros((16,), jnp.int32)
        pltpu.sync_copy(row_v, hist_s.at[pl.ds(sid * RPT, RPT)])
        for j in range(CHUNK // 16):
            ones_v[pl.ds(j * 16, 16)] = jnp.ones((16,), jnp.int32)
        pltpu.sync_copy(epk_ref.at[pl.ds(sid * CPT, CPT)], pk_v)
        plsc.subcore_barrier()

        def step(j, carry):
            for k in range(CHUNK // 16):
                w = pk_v[j, pl.ds(k * 16, 16)]
                uidx_v[pl.ds(k * 16, 16)] = (
                    w >> 16 if high else w & 0xFFFF)
            pltpu.sync_copy(ones_v, hist_s.at[uidx_v], add=True)
            return carry

        lax.fori_loop(0, CPT, step, 0)
        plsc.subcore_barrier()
        pltpu.sync_copy(hist_s.at[pl.ds(sid * RPT, RPT)], row_v)
        pltpu.sync_copy(row_v, out_ref.at[pl.ds(sid * RPT, RPT)])

    @pl.when(cid == 0)
    def _():
        phase(True, cr_ref)

    @pl.when(cid == 1)
    def _():
        phase(False, cc_ref)


@functools.partial(
    pl.kernel,
    out_type=tuple(jax.ShapeDtypeStruct((NPAD, 128), jnp.float32)
                   for _ in range(4)),
    mesh=_MESH,
    scratch_types=[
        pltpu.VMEM((CPT, CHUNK), jnp.int32),
        pltpu.VMEM((NBUF, CHUNK), jnp.int32),
        pltpu.VMEM((NBUF, CHUNK), jnp.int32),
        pltpu.VMEM((NBUF, CHUNK, 128), jnp.float32),
        pltpu.VMEM_SHARED((NPAD, 128), jnp.float32),
        pltpu.SemaphoreType.DMA,
        pltpu.SemaphoreType.DMA,
    ],
)
def _sc_scatter(epk_ref, mil_ref, mir_ref, mol_ref, mor_ref,
                pil_ref, pir_ref, pol_ref, por_ref,
                pk_v, sidx_v, didx_v, g_v, acc_s, sem0, sem1):
    cid = lax.axis_index("c")
    sid = lax.axis_index("s")
    sems = (sem0, sem1)

    def do_phase(m_ref, p_ref, src_high):
        for k in range(RPT // CHUNK):
            off = sid * RPT + k * CHUNK
            pltpu.sync_copy(m_ref.at[pl.ds(off, CHUNK)], g_v.at[k % NBUF])
            pltpu.sync_copy(g_v.at[k % NBUF], acc_s.at[pl.ds(off, CHUNK)])
        plsc.subcore_barrier()

        def unpack_fire(j, b):
            for k in range(CHUNK // 16):
                w = pk_v[j, pl.ds(k * 16, 16)]
                sidx_v[b, pl.ds(k * 16, 16)] = (
                    w >> 16 if src_high else w & 0xFFFF)
                didx_v[b, pl.ds(k * 16, 16)] = (
                    w & 0xFFFF if src_high else w >> 16)
            pltpu.async_copy(m_ref.at[sidx_v.at[b]], g_v.at[b], sems[b])

        for b in range(NBUF):
            unpack_fire(b, b)

        def step(t, carry):
            for b in range(NBUF):
                j = t * NBUF + b
                pltpu.make_async_copy(
                    m_ref.at[sidx_v.at[b]], g_v.at[b], sems[b]).wait()
                pltpu.sync_copy(g_v.at[b], acc_s.at[didx_v.at[b]], add=True)

                @pl.when(j + NBUF < CPT)
                def _():
                    unpack_fire(j + NBUF, b)
            return carry

        lax.fori_loop(0, CPT // NBUF, step, 0)
        plsc.subcore_barrier()
        for k in range(RPT // CHUNK):
            off = sid * RPT + k * CHUNK
            pltpu.sync_copy(acc_s.at[pl.ds(off, CHUNK)], g_v.at[k % NBUF])
            pltpu.sync_copy(g_v.at[k % NBUF], p_ref.at[pl.ds(off, CHUNK)])
        plsc.subcore_barrier()

    pltpu.sync_copy(epk_ref.at[pl.ds(sid * CPT, CPT)], pk_v)

    @pl.when(cid == 0)
    def _():
        do_phase(mil_ref, pil_ref, True)
        do_phase(mol_ref, pol_ref, False)

    @pl.when(cid == 1)
    def _():
        do_phase(mir_ref, pir_ref, True)
        do_phase(mor_ref, por_ref, False)


def _tc_prep_body(x_ref, w_ref, dc_ref, dr_ref, mil, mir, mol, mor, di, do_):
    h = jnp.dot(x_ref[...], w_ref[...], preferred_element_type=jnp.float32)
    din = lax.rsqrt(dc_ref[...].astype(jnp.float32) + 1.0)
    dou = lax.rsqrt(dr_ref[...].astype(jnp.float32) + 1.0)
    mil[...] = h[:, 0:128] * din
    mir[...] = h[:, 128:256] * din
    mol[...] = h[:, 256:384] * dou
    mor[...] = h[:, 384:512] * dou
    di[...] = din
    do_[...] = dou


_tc_prep = pl.pallas_call(
    _tc_prep_body,
    grid=(NPAD // BR,),
    in_specs=[
        pl.BlockSpec((BR, D), lambda i: (i, 0)),
        pl.BlockSpec((D, 2 * D), lambda i: (0, 0)),
        pl.BlockSpec((BR, 1), lambda i: (i, 0)),
        pl.BlockSpec((BR, 1), lambda i: (i, 0)),
    ],
    out_specs=[
        pl.BlockSpec((BR, 128), lambda i: (i, 0)),
        pl.BlockSpec((BR, 128), lambda i: (i, 0)),
        pl.BlockSpec((BR, 128), lambda i: (i, 0)),
        pl.BlockSpec((BR, 128), lambda i: (i, 0)),
        pl.BlockSpec((BR, 1), lambda i: (i, 0)),
        pl.BlockSpec((BR, 1), lambda i: (i, 0)),
    ],
    out_shape=[jax.ShapeDtypeStruct((NPAD, 128), jnp.float32)] * 4
    + [jax.ShapeDtypeStruct((NPAD, 1), jnp.float32)] * 2,
)


def _tc_final_body(pil, pir, pol, por, di, do_, wfc, bi, bo, bf, out):
    pin = jnp.concatenate([pil[...], pir[...]], axis=1) * di[...]
    pou = jnp.concatenate([pol[...], por[...]], axis=1) * do_[...]
    big = jnp.concatenate([pin, pou], axis=1)
    y = jnp.dot(big, wfc[...], preferred_element_type=jnp.float32)
    bias = (jnp.dot(bi[...], wfc[0:D, :], preferred_element_type=jnp.float32)
            + jnp.dot(bo[...], wfc[D:2 * D, :],
                      preferred_element_type=jnp.float32)
            + bf[...])
    out[...] = jnp.maximum(y + bias, 0.0)


_tc_final = pl.pallas_call(
    _tc_final_body,
    grid=(NPAD // BR,),
    in_specs=[
        pl.BlockSpec((BR, 128), lambda i: (i, 0)),
        pl.BlockSpec((BR, 128), lambda i: (i, 0)),
        pl.BlockSpec((BR, 128), lambda i: (i, 0)),
        pl.BlockSpec((BR, 128), lambda i: (i, 0)),
        pl.BlockSpec((BR, 1), lambda i: (i, 0)),
        pl.BlockSpec((BR, 1), lambda i: (i, 0)),
        pl.BlockSpec((2 * D, D), lambda i: (0, 0)),
        pl.BlockSpec((1, D), lambda i: (0, 0)),
        pl.BlockSpec((1, D), lambda i: (0, 0)),
        pl.BlockSpec((1, D), lambda i: (0, 0)),
    ],
    out_specs=pl.BlockSpec((BR, D), lambda i: (i, 0)),
    out_shape=jax.ShapeDtypeStruct((NPAD, D), jnp.float32),
)


def kernel(x, edge_index, W_in, b_in, W_out, b_out, W_fc, b_fc):
    ei = edge_index.astype(jnp.int32)
    pad_idx = N + (jnp.arange(EPAD - E, dtype=jnp.int32) % (NPAD - N))
    erow = jnp.concatenate([ei[0], pad_idx])
    ecol = jnp.concatenate([ei[1], pad_idx])
    epk = ((erow << 16) | ecol).reshape(NCH, CHUNK)
    xp = jnp.pad(x, ((0, NPAD - N), (0, 0)))
    Wcat = jnp.concatenate([W_in, W_out], axis=1)
    cnt_row, cnt_col = _sc_degrees(epk)
    mil, mir, mol, mor, din, dou = _tc_prep(
        xp, Wcat, cnt_col[:, None], cnt_row[:, None])
    pil, pir, pol, por = _sc_scatter(epk, mil, mir, mol, mor)
    y = _tc_final(pil, pir, pol, por, din, dou, W_fc,
                  b_in[None, :], b_out[None, :], b_fc[None, :])
    return y[:N]


# speedup vs baseline: 23.1718x; 1.2125x over previous
import functools

import jax
import jax.numpy as jnp
from jax import lax
from jax.experimental import pallas as pl
from jax.experimental.pallas import tpu as pltpu
from jax.experimental.pallas import tpu_sc as plsc

N = 10000
E = 160000
D = 256
NPAD = 10240
RPT = NPAD // 16
CHUNK = 64
EPAD = 163840
NCH = EPAD // CHUNK
CPT = NCH // 16
NBUF = 3
NFT = CPT // NBUF
REM = CPT - NFT * NBUF
BR = 1024

_MESH = plsc.VectorSubcoreMesh(core_axis_name="c", subcore_axis_name="s")


@functools.partial(
    pl.kernel,
    out_type=(jax.ShapeDtypeStruct((NPAD,), jnp.int32),
              jax.ShapeDtypeStruct((NPAD,), jnp.int32)),
    mesh=_MESH,
    scratch_types=[
        pltpu.VMEM((CPT, CHUNK), jnp.int32),
        pltpu.VMEM((2 * CHUNK,), jnp.int32),
        pltpu.VMEM((2 * CHUNK,), jnp.int32),
        pltpu.VMEM((RPT,), jnp.int32),
        pltpu.VMEM_SHARED((NPAD,), jnp.int32),
    ],
)
def _sc_degrees(epk_ref, cr_ref, cc_ref, pk_v, uidx_v, ones_v, row_v, hist_s):
    cid = lax.axis_index("c")
    sid = lax.axis_index("s")

    def phase(high, out_ref):
        for j in range(RPT // 16):
            row_v[pl.ds(j * 16, 16)] = jnp.zeros((16,), jnp.int32)
        pltpu.sync_copy(row_v, hist_s.at[pl.ds(sid * RPT, RPT)])
        for j in range(2 * CHUNK // 16):
            ones_v[pl.ds(j * 16, 16)] = jnp.ones((16,), jnp.int32)
        pltpu.sync_copy(epk_ref.at[pl.ds(sid * CPT, CPT)], pk_v)
        plsc.subcore_barrier()

        def step(t, carry):
            for r in range(2):
                for k in range(CHUNK // 16):
                    w = pk_v[t * 2 + r, pl.ds(k * 16, 16)]
                    uidx_v[pl.ds(r * CHUNK + k * 16, 16)] = (
                        w >> 16 if high else w & 0xFFFF)
            pltpu.sync_copy(ones_v, hist_s.at[uidx_v], add=True)
            return carry

        lax.fori_loop(0, CPT // 2, step, 0)
        plsc.subcore_barrier()
        pltpu.sync_copy(hist_s.at[pl.ds(sid * RPT, RPT)], row_v)
        pltpu.sync_copy(row_v, out_ref.at[pl.ds(sid * RPT, RPT)])

    @pl.when(cid == 0)
    def _():
        phase(True, cr_ref)

    @pl.when(cid == 1)
    def _():
        phase(False, cc_ref)


@functools.partial(
    pl.kernel,
    out_type=tuple(jax.ShapeDtypeStruct((NPAD, 128), jnp.float32)
                   for _ in range(4)),
    mesh=_MESH,
    scratch_types=[
        pltpu.VMEM((CPT, CHUNK), jnp.int32),
        pltpu.VMEM((NBUF, CHUNK), jnp.int32),
        pltpu.VMEM((NBUF, CHUNK), jnp.int32),
        pltpu.VMEM((NBUF, CHUNK, 128), jnp.float32),
        pltpu.VMEM_SHARED((NPAD, 128), jnp.float32),
        pltpu.SemaphoreType.DMA,
        pltpu.SemaphoreType.DMA,
        pltpu.SemaphoreType.DMA,
    ],
)
def _sc_scatter(epk_ref, mil_ref, mir_ref, mol_ref, mor_ref,
                pil_ref, pir_ref, pol_ref, por_ref,
                pk_v, sidx_v, didx_v, g_v, acc_s, sem0, sem1, sem2):
    cid = lax.axis_index("c")
    sid = lax.axis_index("s")
    sems = (sem0, sem1, sem2)

    def do_phase(m_ref, p_ref, src_high):
        for k in range(RPT // CHUNK):
            off = sid * RPT + k * CHUNK
            pltpu.sync_copy(m_ref.at[pl.ds(off, CHUNK)], g_v.at[k % NBUF])
            pltpu.sync_copy(g_v.at[k % NBUF], acc_s.at[pl.ds(off, CHUNK)])
        plsc.subcore_barrier()

        def unpack_fire(j, b):
            for k in range(CHUNK // 16):
                w = pk_v[j, pl.ds(k * 16, 16)]
                sidx_v[b, pl.ds(k * 16, 16)] = (
                    w >> 16 if src_high else w & 0xFFFF)
                didx_v[b, pl.ds(k * 16, 16)] = (
                    w & 0xFFFF if src_high else w >> 16)
            pltpu.async_copy(m_ref.at[sidx_v.at[b]], g_v.at[b], sems[b])

        for b in range(NBUF):
            unpack_fire(b, b)

        def step(t, carry):
            for b in range(NBUF):
                j = t * NBUF + b
                pltpu.make_async_copy(
                    m_ref.at[sidx_v.at[b]], g_v.at[b], sems[b]).wait()
                pltpu.sync_copy(g_v.at[b], acc_s.at[didx_v.at[b]], add=True)

                @pl.when(j + NBUF < CPT)
                def _():
                    unpack_fire(j + NBUF, b)
            return carry

        lax.fori_loop(0, NFT, step, 0)
        for r in range(REM):
            jr = NFT * NBUF + r
            br = jr % NBUF
            pltpu.make_async_copy(
                m_ref.at[sidx_v.at[br]], g_v.at[br], sems[br]).wait()
            pltpu.sync_copy(g_v.at[br], acc_s.at[didx_v.at[br]], add=True)
        plsc.subcore_barrier()
        for k in range(RPT // CHUNK):
            off = sid * RPT + k * CHUNK
            pltpu.sync_copy(acc_s.at[pl.ds(off, CHUNK)], g_v.at[k % NBUF])
            pltpu.sync_copy(g_v.at[k % NBUF], p_ref.at[pl.ds(off, CHUNK)])
        plsc.subcore_barrier()

    pltpu.sync_copy(epk_ref.at[pl.ds(sid * CPT, CPT)], pk_v)

    @pl.when(cid == 0)
    def _():
        do_phase(mil_ref, pil_ref, True)
        do_phase(mol_ref, pol_ref, False)

    @pl.when(cid == 1)
    def _():
        do_phase(mir_ref, pir_ref, True)
        do_phase(mor_ref, por_ref, False)


def _tc_prep_body(x_ref, w_ref, dc_ref, dr_ref, mil, mir, mol, mor, di, do_):
    h = jnp.dot(x_ref[...], w_ref[...], preferred_element_type=jnp.float32)
    din = lax.rsqrt(dc_ref[...].astype(jnp.float32) + 1.0)
    dou = lax.rsqrt(dr_ref[...].astype(jnp.float32) + 1.0)
    mil[...] = h[:, 0:128] * din
    mir[...] = h[:, 128:256] * din
    mol[...] = h[:, 256:384] * dou
    mor[...] = h[:, 384:512] * dou
    di[...] = din
    do_[...] = dou


_tc_prep = pl.pallas_call(
    _tc_prep_body,
    grid=(NPAD // BR,),
    in_specs=[
        pl.BlockSpec((BR, D), lambda i: (i, 0)),
        pl.BlockSpec((D, 2 * D), lambda i: (0, 0)),
        pl.BlockSpec((BR, 1), lambda i: (i, 0)),
        pl.BlockSpec((BR, 1), lambda i: (i, 0)),
    ],
    out_specs=[
        pl.BlockSpec((BR, 128), lambda i: (i, 0)),
        pl.BlockSpec((BR, 128), lambda i: (i, 0)),
        pl.BlockSpec((BR, 128), lambda i: (i, 0)),
        pl.BlockSpec((BR, 128), lambda i: (i, 0)),
        pl.BlockSpec((BR, 1), lambda i: (i, 0)),
        pl.BlockSpec((BR, 1), lambda i: (i, 0)),
    ],
    out_shape=[jax.ShapeDtypeStruct((NPAD, 128), jnp.float32)] * 4
    + [jax.ShapeDtypeStruct((NPAD, 1), jnp.float32)] * 2,
)


def _tc_final_body(pil, pir, pol, por, di, do_, wfc, bi, bo, bf, out):
    pin = jnp.concatenate([pil[...], pir[...]], axis=1) * di[...]
    pou = jnp.concatenate([pol[...], por[...]], axis=1) * do_[...]
    big = jnp.concatenate([pin, pou], axis=1)
    y = jnp.dot(big, wfc[...], preferred_element_type=jnp.float32)
    bias = (jnp.dot(bi[...], wfc[0:D, :], preferred_element_type=jnp.float32)
            + jnp.dot(bo[...], wfc[D:2 * D, :],
                      preferred_element_type=jnp.float32)
            + bf[...])
    out[...] = jnp.maximum(y + bias, 0.0)


_tc_final = pl.pallas_call(
    _tc_final_body,
    grid=(NPAD // BR,),
    in_specs=[
        pl.BlockSpec((BR, 128), lambda i: (i, 0)),
        pl.BlockSpec((BR, 128), lambda i: (i, 0)),
        pl.BlockSpec((BR, 128), lambda i: (i, 0)),
        pl.BlockSpec((BR, 128), lambda i: (i, 0)),
        pl.BlockSpec((BR, 1), lambda i: (i, 0)),
        pl.BlockSpec((BR, 1), lambda i: (i, 0)),
        pl.BlockSpec((2 * D, D), lambda i: (0, 0)),
        pl.BlockSpec((1, D), lambda i: (0, 0)),
        pl.BlockSpec((1, D), lambda i: (0, 0)),
        pl.BlockSpec((1, D), lambda i: (0, 0)),
    ],
    out_specs=pl.BlockSpec((BR, D), lambda i: (i, 0)),
    out_shape=jax.ShapeDtypeStruct((NPAD, D), jnp.float32),
)


def kernel(x, edge_index, W_in, b_in, W_out, b_out, W_fc, b_fc):
    ei = edge_index.astype(jnp.int32)
    pad_idx = N + (jnp.arange(EPAD - E, dtype=jnp.int32) % (NPAD - N))
    erow = jnp.concatenate([ei[0], pad_idx])
    ecol = jnp.concatenate([ei[1], pad_idx])
    epk = ((erow << 16) | ecol).reshape(NCH, CHUNK)
    xp = jnp.pad(x, ((0, NPAD - N), (0, 0)))
    Wcat = jnp.concatenate([W_in, W_out], axis=1)
    cnt_row, cnt_col = _sc_degrees(epk)
    mil, mir, mol, mor, din, dou = _tc_prep(
        xp, Wcat, cnt_col[:, None], cnt_row[:, None])
    pil, pir, pol, por = _sc_scatter(epk, mil, mir, mol, mor)
    y = _tc_final(pil, pir, pol, por, din, dou, W_fc,
                  b_in[None, :], b_out[None, :], b_fc[None, :])
    return y[:N]


# speedup vs baseline: 23.9687x; 1.0344x over previous
import functools

import jax
import jax.numpy as jnp
from jax import lax
from jax.experimental import pallas as pl
from jax.experimental.pallas import tpu as pltpu
from jax.experimental.pallas import tpu_sc as plsc

N = 10000
E = 160000
D = 256
NPAD = 10240
RPT = NPAD // 16
CHUNK = 64
EPAD = 163840
NCH = EPAD // CHUNK
CPT = NCH // 16
NBUF = 3
NFT = CPT // NBUF
REM = CPT - NFT * NBUF
BR = 1024

_MESH = plsc.VectorSubcoreMesh(core_axis_name="c", subcore_axis_name="s")


@functools.partial(
    pl.kernel,
    out_type=(jax.ShapeDtypeStruct((NPAD,), jnp.int32),
              jax.ShapeDtypeStruct((NPAD,), jnp.int32)),
    mesh=_MESH,
    scratch_types=[
        pltpu.VMEM((CPT, CHUNK), jnp.int32),
        pltpu.VMEM((2 * CHUNK,), jnp.int32),
        pltpu.VMEM((2 * CHUNK,), jnp.int32),
        pltpu.VMEM((RPT,), jnp.int32),
        pltpu.VMEM_SHARED((NPAD,), jnp.int32),
    ],
)
def _sc_degrees(epk_ref, cr_ref, cc_ref, pk_v, uidx_v, ones_v, row_v, hist_s):
    cid = lax.axis_index("c")
    sid = lax.axis_index("s")

    def phase(high, out_ref):
        for j in range(RPT // 16):
            row_v[pl.ds(j * 16, 16)] = jnp.zeros((16,), jnp.int32)
        pltpu.sync_copy(row_v, hist_s.at[pl.ds(sid * RPT, RPT)])
        for j in range(2 * CHUNK // 16):
            ones_v[pl.ds(j * 16, 16)] = jnp.ones((16,), jnp.int32)
        pltpu.sync_copy(epk_ref.at[pl.ds(sid * CPT, CPT)], pk_v)
        plsc.subcore_barrier()

        def step(t, carry):
            for r in range(2):
                for k in range(CHUNK // 16):
                    w = pk_v[t * 2 + r, pl.ds(k * 16, 16)]
                    uidx_v[pl.ds(r * CHUNK + k * 16, 16)] = (
                        w >> 16 if high else w & 0xFFFF)
            pltpu.sync_copy(ones_v, hist_s.at[uidx_v], add=True)
            return carry

        lax.fori_loop(0, CPT // 2, step, 0)
        plsc.subcore_barrier()
        pltpu.sync_copy(hist_s.at[pl.ds(sid * RPT, RPT)], row_v)
        pltpu.sync_copy(row_v, out_ref.at[pl.ds(sid * RPT, RPT)])

    @pl.when(cid == 0)
    def _():
        phase(True, cr_ref)

    @pl.when(cid == 1)
    def _():
        phase(False, cc_ref)


@functools.partial(
    pl.kernel,
    out_type=tuple(jax.ShapeDtypeStruct((NPAD, 128), jnp.float32)
                   for _ in range(4)),
    mesh=_MESH,
    scratch_types=[
        pltpu.VMEM((CPT, CHUNK), jnp.int32),
        pltpu.VMEM((NBUF, CHUNK), jnp.int32),
        pltpu.VMEM((NBUF, CHUNK), jnp.int32),
        pltpu.VMEM((NBUF, CHUNK, 128), jnp.float32),
        pltpu.VMEM_SHARED((NPAD, 128), jnp.float32),
        pltpu.SemaphoreType.DMA,
        pltpu.SemaphoreType.DMA,
        pltpu.SemaphoreType.DMA,
    ],
)
def _sc_scatter(epk_ref, mil_ref, mir_ref, mol_ref, mor_ref,
                pil_ref, pir_ref, pol_ref, por_ref,
                pk_v, sidx_v, didx_v, g_v, acc_s, sem0, sem1, sem2):
    cid = lax.axis_index("c")
    sid = lax.axis_index("s")
    sems = (sem0, sem1, sem2)

    def do_phase(m_ref, p_ref, src_high):
        for k in range(RPT // CHUNK):
            off = sid * RPT + k * CHUNK
            pltpu.sync_copy(m_ref.at[pl.ds(off, CHUNK)], g_v.at[k % NBUF])
            pltpu.sync_copy(g_v.at[k % NBUF], acc_s.at[pl.ds(off, CHUNK)])
        plsc.subcore_barrier()

        def unpack_fire(j, b):
            for k in range(CHUNK // 16):
                w = pk_v[j, pl.ds(k * 16, 16)]
                sidx_v[b, pl.ds(k * 16, 16)] = (
                    w >> 16 if src_high else w & 0xFFFF)
                didx_v[b, pl.ds(k * 16, 16)] = (
                    w & 0xFFFF if src_high else w >> 16)
            pltpu.async_copy(m_ref.at[sidx_v.at[b]], g_v.at[b], sems[b])

        for b in range(NBUF):
            unpack_fire(b, b)

        def step(t, carry):
            for b in range(NBUF):
                j = t * NBUF + b
                pltpu.make_async_copy(
                    m_ref.at[sidx_v.at[b]], g_v.at[b], sems[b]).wait()
                pltpu.sync_copy(g_v.at[b], acc_s.at[didx_v.at[b]], add=True)

                @pl.when(j + NBUF < CPT)
                def _():
                    unpack_fire(j + NBUF, b)
            return carry

        lax.fori_loop(0, NFT, step, 0)
        for r in range(REM):
            jr = NFT * NBUF + r
            br = jr % NBUF
            pltpu.make_async_copy(
                m_ref.at[sidx_v.at[br]], g_v.at[br], sems[br]).wait()
            pltpu.sync_copy(g_v.at[br], acc_s.at[didx_v.at[br]], add=True)
        plsc.subcore_barrier()
        for k in range(RPT // CHUNK):
            off = sid * RPT + k * CHUNK
            pltpu.sync_copy(acc_s.at[pl.ds(off, CHUNK)], g_v.at[k % NBUF])
            pltpu.sync_copy(g_v.at[k % NBUF], p_ref.at[pl.ds(off, CHUNK)])
        plsc.subcore_barrier()

    pltpu.sync_copy(epk_ref.at[pl.ds(sid * CPT, CPT)], pk_v)

    @pl.when(cid == 0)
    def _():
        do_phase(mil_ref, pil_ref, True)
        do_phase(mol_ref, pol_ref, False)

    @pl.when(cid == 1)
    def _():
        do_phase(mir_ref, pir_ref, True)
        do_phase(mor_ref, por_ref, False)


def _tc_prep_body(x_ref, w_ref, dc_ref, dr_ref, mil, mir, mol, mor, di, do_):
    h = jnp.dot(x_ref[...], w_ref[...], preferred_element_type=jnp.float32)
    din = lax.rsqrt(dc_ref[...].astype(jnp.float32) + 1.0)
    dou = lax.rsqrt(dr_ref[...].astype(jnp.float32) + 1.0)
    mil[...] = h[:, 0:128] * din
    mir[...] = h[:, 128:256] * din
    mol[...] = h[:, 256:384] * dou
    mor[...] = h[:, 384:512] * dou
    di[...] = din
    do_[...] = dou


_tc_prep = pl.pallas_call(
    _tc_prep_body,
    grid=(NPAD // BR,),
    in_specs=[
        pl.BlockSpec((BR, D), lambda i: (i, 0)),

        pl.BlockSpec((D, 2 * D), lambda i: (0, 0)),
        pl.BlockSpec((BR, 1), lambda i: (i, 0)),
        pl.BlockSpec((BR, 1), lambda i: (i, 0)),
    ],
    out_specs=[
        pl.BlockSpec((BR, 128), lambda i: (i, 0)),
        pl.BlockSpec((BR, 128), lambda i: (i, 0)),
        pl.BlockSpec((BR, 128), lambda i: (i, 0)),
        pl.BlockSpec((BR, 128), lambda i: (i, 0)),
        pl.BlockSpec((BR, 1), lambda i: (i, 0)),
        pl.BlockSpec((BR, 1), lambda i: (i, 0)),
    ],
    out_shape=[jax.ShapeDtypeStruct((NPAD, 128), jnp.float32)] * 4
    + [jax.ShapeDtypeStruct((NPAD, 1), jnp.float32)] * 2,
)


def _tc_final_body(pil, pir, pol, por, di, do_, wfc, bi, bo, bf, out):
    pin = jnp.concatenate([pil[...], pir[...]], axis=1) * di[...]
    pou = jnp.concatenate([pol[...], por[...]], axis=1) * do_[...]
    big = jnp.concatenate([pin, pou], axis=1)
    y = jnp.dot(big, wfc[...], preferred_element_type=jnp.float32)
    bias = (jnp.dot(bi[...], wfc[0:D, :], preferred_element_type=jnp.float32)
            + jnp.dot(bo[...], wfc[D:2 * D, :],
                      preferred_element_type=jnp.float32)
            + bf[...])
    out[...] = jnp.maximum(y + bias, 0.0)


_tc_final = pl.pallas_call(
    _tc_final_body,
    grid=(NPAD // BR,),
    in_specs=[
        pl.BlockSpec((BR, 128), lambda i: (i, 0)),
        pl.BlockSpec((BR, 128), lambda i: (i, 0)),
        pl.BlockSpec((BR, 128), lambda i: (i, 0)),
        pl.BlockSpec((BR, 128), lambda i: (i, 0)),
        pl.BlockSpec((BR, 1), lambda i: (i, 0)),
        pl.BlockSpec((BR, 1), lambda i: (i, 0)),
        pl.BlockSpec((2 * D, D), lambda i: (0, 0)),
        pl.BlockSpec((1, D), lambda i: (0, 0)),
        pl.BlockSpec((1, D), lambda i: (0, 0)),
        pl.BlockSpec((1, D), lambda i: (0, 0)),
    ],
    out_specs=pl.BlockSpec((BR, D), lambda i: (i, 0)),
    out_shape=jax.ShapeDtypeStruct((N, D), jnp.float32),
)


def kernel(x, edge_index, W_in, b_in, W_out, b_out, W_fc, b_fc):
    ei = edge_index.astype(jnp.int32)
    pad_idx = N + (jnp.arange(EPAD - E, dtype=jnp.int32) % (NPAD - N))
    erow = jnp.concatenate([ei[0], pad_idx])
    ecol = jnp.concatenate([ei[1], pad_idx])
    epk = ((erow << 16) | ecol).reshape(NCH, CHUNK)
    Wcat = jnp.concatenate([W_in, W_out], axis=1)
    cnt_row, cnt_col = _sc_degrees(epk)
    mil, mir, mol, mor, din, dou = _tc_prep(
        x, Wcat, cnt_col[:, None], cnt_row[:, None])
    pil, pir, pol, por = _sc_scatter(epk, mil, mir, mol, mor)
    return _tc_final(pil, pir, pol, por, din, dou, W_fc,
                     b_in[None, :], b_out[None, :], b_fc[None, :])


# speedup vs baseline: 23.9920x; 1.0010x over previous
import functools

import jax
import jax.numpy as jnp
from jax import lax
from jax.experimental import pallas as pl
from jax.experimental.pallas import tpu as pltpu
from jax.experimental.pallas import tpu_sc as plsc

N = 10000
E = 160000
D = 256
NPAD = 10240
RPT = NPAD // 16
CHUNK = 64
EPAD = 163840
NCH = EPAD // CHUNK
CPT = NCH // 16
NBUF = 3
NFT = CPT // NBUF
REM = CPT - NFT * NBUF
BR = 1024

_MESH = plsc.VectorSubcoreMesh(core_axis_name="c", subcore_axis_name="s")


@functools.partial(
    pl.kernel,
    out_type=(jax.ShapeDtypeStruct((NPAD,), jnp.int32),
              jax.ShapeDtypeStruct((NPAD,), jnp.int32)),
    mesh=_MESH,
    scratch_types=[
        pltpu.VMEM((CPT, CHUNK), jnp.int32),
        pltpu.VMEM((2 * CHUNK,), jnp.int32),
        pltpu.VMEM((2 * CHUNK,), jnp.int32),
        pltpu.VMEM((RPT,), jnp.int32),
        pltpu.VMEM_SHARED((NPAD,), jnp.int32),
    ],
)
def _sc_degrees(epk_ref, cr_ref, cc_ref, pk_v, uidx_v, ones_v, row_v, hist_s):
    cid = lax.axis_index("c")
    sid = lax.axis_index("s")

    def phase(high, out_ref):
        for j in range(RPT // 16):
            row_v[pl.ds(j * 16, 16)] = jnp.zeros((16,), jnp.int32)
        pltpu.sync_copy(row_v, hist_s.at[pl.ds(sid * RPT, RPT)])
        for j in range(2 * CHUNK // 16):
            ones_v[pl.ds(j * 16, 16)] = jnp.ones((16,), jnp.int32)
        pltpu.sync_copy(epk_ref.at[pl.ds(sid * CPT, CPT)], pk_v)
        plsc.subcore_barrier()

        def step(t, carry):
            for r in range(2):
                for k in range(CHUNK // 16):
                    w = pk_v[t * 2 + r, pl.ds(k * 16, 16)]
                    uidx_v[pl.ds(r * CHUNK + k * 16, 16)] = (
                        w >> 16 if high else w & 0xFFFF)
            pltpu.sync_copy(ones_v, hist_s.at[uidx_v], add=True)
            return carry

        lax.fori_loop(0, CPT // 2, step, 0)
        plsc.subcore_barrier()
        pltpu.sync_copy(hist_s.at[pl.ds(sid * RPT, RPT)], row_v)
        pltpu.sync_copy(row_v, out_ref.at[pl.ds(sid * RPT, RPT)])

    @pl.when(cid == 0)
    def _():
        phase(True, cr_ref)

    @pl.when(cid == 1)
    def _():
        phase(False, cc_ref)


@functools.partial(
    pl.kernel,
    out_type=tuple(jax.ShapeDtypeStruct((NPAD, 128), jnp.float32)
                   for _ in range(4)),
    mesh=_MESH,
    scratch_types=[
        pltpu.VMEM((CPT, CHUNK), jnp.int32),
        pltpu.VMEM((NBUF, CHUNK), jnp.int32),
        pltpu.VMEM((NBUF, CHUNK), jnp.int32),
        pltpu.VMEM((NBUF, CHUNK, 128), jnp.float32),
        pltpu.VMEM_SHARED((NPAD, 128), jnp.float32),
        pltpu.SemaphoreType.DMA,
        pltpu.SemaphoreType.DMA,
        pltpu.SemaphoreType.DMA,
        pltpu.SemaphoreType.DMA,
        pltpu.SemaphoreType.DMA,
        pltpu.SemaphoreType.DMA,
    ],
)
def _sc_scatter(epk_ref, mil_ref, mir_ref, mol_ref, mor_ref,
                pil_ref, pir_ref, pol_ref, por_ref,
                pk_v, sidx_v, didx_v, g_v, acc_s,
                sem0, sem1, sem2, sem3, sem4, sem5):
    cid = lax.axis_index("c")
    sid = lax.axis_index("s")
    sems = (sem0, sem1, sem2)
    ssems = (sem3, sem4, sem5)

    def do_phase(m_ref, p_ref, src_high):
        for k in range(RPT // CHUNK):
            off = sid * RPT + k * CHUNK
            pltpu.sync_copy(m_ref.at[pl.ds(off, CHUNK)], g_v.at[k % NBUF])
            pltpu.sync_copy(g_v.at[k % NBUF], acc_s.at[pl.ds(off, CHUNK)])
        plsc.subcore_barrier()

        def unpack_fire(j, b, first):
            if not first:
                pltpu.make_async_copy(
                    g_v.at[b], acc_s.at[didx_v.at[b]], ssems[b]).wait()
            for k in range(CHUNK // 16):
                w = pk_v[j, pl.ds(k * 16, 16)]
                sidx_v[b, pl.ds(k * 16, 16)] = (
                    w >> 16 if src_high else w & 0xFFFF)
                didx_v[b, pl.ds(k * 16, 16)] = (
                    w & 0xFFFF if src_high else w >> 16)
            pltpu.async_copy(m_ref.at[sidx_v.at[b]], g_v.at[b], sems[b])

        for b in range(NBUF):
            unpack_fire(b, b, True)

        def step(t, carry):
            for b in range(NBUF):
                j = t * NBUF + b
                pltpu.make_async_copy(
                    m_ref.at[sidx_v.at[b]], g_v.at[b], sems[b]).wait()
                pltpu.async_copy(
                    g_v.at[b], acc_s.at[didx_v.at[b]], ssems[b], add=True)

                @pl.when(j + NBUF < CPT)
                def _():
                    unpack_fire(j + NBUF, b, False)
            return carry

        lax.fori_loop(0, NFT, step, 0)
        for r in range(REM):
            jr = NFT * NBUF + r
            br = jr % NBUF
            pltpu.make_async_copy(
                m_ref.at[sidx_v.at[br]], g_v.at[br], sems[br]).wait()
            pltpu.async_copy(
                g_v.at[br], acc_s.at[didx_v.at[br]], ssems[br], add=True)
        for b in range(NBUF):
            pltpu.make_async_copy(
                g_v.at[b], acc_s.at[didx_v.at[b]], ssems[b]).wait()
        plsc.subcore_barrier()
        for k in range(RPT // CHUNK):
            off = sid * RPT + k * CHUNK
            pltpu.sync_copy(acc_s.at[pl.ds(off, CHUNK)], g_v.at[k % NBUF])
            pltpu.sync_copy(g_v.at[k % NBUF], p_ref.at[pl.ds(off, CHUNK)])
        plsc.subcore_barrier()

    pltpu.sync_copy(epk_ref.at[pl.ds(sid * CPT, CPT)], pk_v)

    @pl.when(cid == 0)
    def _():
        do_phase(mil_ref, pil_ref, True)
        do_phase(mol_ref, pol_ref, False)

    @pl.when(cid == 1)
    def _():
        do_phase(mir_ref, pir_ref, True)
        do_phase(mor_ref, por_ref, False)


def _tc_prep_body(x_ref, w_ref, dc_ref, dr_ref, mil, mir, mol, mor, di, do_):
    h = jnp.dot(x_ref[...], w_ref[...], preferred_element_type=jnp.float32)
    din = lax.rsqrt(dc_ref[...].astype(jnp.float32) + 1.0)
    dou = lax.rsqrt(dr_ref[...].astype(jnp.float32) + 1.0)
    mil[...] = h[:, 0:128] * din
    mir[...] = h[:, 128:256] * din
    mol[...] = h[:, 256:384] * dou
    mor[...] = h[:, 384:512] * dou
    di[...] = din
    do_[...] = dou


_tc_prep = pl.pallas_call(
    _tc_prep_body,
    grid=(NPAD // BR,),
    in_specs=[
        pl.BlockSpec((BR, D), lambda i: (i, 0)),

        pl.BlockSpec((D, 2 * D), lambda i: (0, 0)),
        pl.BlockSpec((BR, 1), lambda i: (i, 0)),
        pl.BlockSpec((BR, 1), lambda i: (i, 0)),
    ],
    out_specs=[
        pl.BlockSpec((BR, 128), lambda i: (i, 0)),
        pl.BlockSpec((BR, 128), lambda i: (i, 0)),
        pl.BlockSpec((BR, 128), lambda i: (i, 0)),
        pl.BlockSpec((BR, 128), lambda i: (i, 0)),
        pl.BlockSpec((BR, 1), lambda i: (i, 0)),
        pl.BlockSpec((BR, 1), lambda i: (i, 0)),
    ],
    out_shape=[jax.ShapeDtypeStruct((NPAD, 128), jnp.float32)] * 4
    + [jax.ShapeDtypeStruct((NPAD, 1), jnp.float32)] * 2,
)


def _tc_final_body(pil, pir, pol, por, di, do_, wfc, bi, bo, bf, out):
    pin = jnp.concatenate([pil[...], pir[...]], axis=1) * di[...]
    pou = jnp.concatenate([pol[...], por[...]], axis=1) * do_[...]
    big = jnp.concatenate([pin, pou], axis=1)
    y = jnp.dot(big, wfc[...], preferred_element_type=jnp.float32)
    bias = (jnp.dot(bi[...], wfc[0:D, :], preferred_element_type=jnp.float32)
            + jnp.dot(bo[...], wfc[D:2 * D, :],
                      preferred_element_type=jnp.float32)
            + bf[...])
    out[...] = jnp.maximum(y + bias, 0.0)


_tc_final = pl.pallas_call(
    _tc_final_body,
    grid=(NPAD // BR,),
    in_specs=[
        pl.BlockSpec((BR, 128), lambda i: (i, 0)),
        pl.BlockSpec((BR, 128), lambda i: (i, 0)),
        pl.BlockSpec((BR, 128), lambda i: (i, 0)),
        pl.BlockSpec((BR, 128), lambda i: (i, 0)),
        pl.BlockSpec((BR, 1), lambda i: (i, 0)),
        pl.BlockSpec((BR, 1), lambda i: (i, 0)),
        pl.BlockSpec((2 * D, D), lambda i: (0, 0)),
        pl.BlockSpec((1, D), lambda i: (0, 0)),
        pl.BlockSpec((1, D), lambda i: (0, 0)),
        pl.BlockSpec((1, D), lambda i: (0, 0)),
    ],
    out_specs=pl.BlockSpec((BR, D), lambda i: (i, 0)),
    out_shape=jax.ShapeDtypeStruct((N, D), jnp.float32),
)


def kernel(x, edge_index, W_in, b_in, W_out, b_out, W_fc, b_fc):
    ei = edge_index.astype(jnp.int32)
    pad_idx = N + (jnp.arange(EPAD - E, dtype=jnp.int32) % (NPAD - N))
    erow = jnp.concatenate([ei[0], pad_idx])
    ecol = jnp.concatenate([ei[1], pad_idx])
    epk = ((erow << 16) | ecol).reshape(NCH, CHUNK)
    Wcat = jnp.concatenate([W_in, W_out], axis=1)
    cnt_row, cnt_col = _sc_degrees(epk)
    mil, mir, mol, mor, din, dou = _tc_prep(
        x, Wcat, cnt_col[:, None], cnt_row[:, None])
    pil, pir, pol, por = _sc_scatter(epk, mil, mir, mol, mor)
    return _tc_final(pil, pir, pol, por, din, dou, W_fc,
                     b_in[None, :], b_out[None, :], b_fc[None, :])


# speedup vs baseline: 25.6602x; 1.0695x over previous
import functools

import jax
import jax.numpy as jnp
from jax import lax
from jax.experimental import pallas as pl
from jax.experimental.pallas import tpu as pltpu
from jax.experimental.pallas import tpu_sc as plsc

N = 10000
E = 160000
D = 256
NPAD = 10240
RPT = NPAD // 16
CHUNK = 64
EPAD = 163840
NCH = EPAD // CHUNK
CPT = NCH // 16
NBUF = 3
NFT = CPT // NBUF
REM = CPT - NFT * NBUF
BR = 1024

_MESH = plsc.VectorSubcoreMesh(core_axis_name="c", subcore_axis_name="s")


@functools.partial(
    pl.kernel,
    out_type=(jax.ShapeDtypeStruct((NPAD,), jnp.int32),
              jax.ShapeDtypeStruct((NPAD,), jnp.int32)),
    mesh=_MESH,
    scratch_types=[
        pltpu.VMEM((CPT, CHUNK), jnp.int32),
        pltpu.VMEM((2 * CHUNK,), jnp.int32),
        pltpu.VMEM((2 * CHUNK,), jnp.int32),
        pltpu.VMEM((RPT,), jnp.int32),
        pltpu.VMEM_SHARED((NPAD,), jnp.int32),
    ],
)
def _sc_degrees(epk_ref, cr_ref, cc_ref, pk_v, uidx_v, ones_v, row_v, hist_s):
    cid = lax.axis_index("c")
    sid = lax.axis_index("s")

    def phase(high, out_ref):
        for j in range(RPT // 16):
            row_v[pl.ds(j * 16, 16)] = jnp.zeros((16,), jnp.int32)
        pltpu.sync_copy(row_v, hist_s.at[pl.ds(sid * RPT, RPT)])
        for j in range(2 * CHUNK // 16):
            ones_v[pl.ds(j * 16, 16)] = jnp.ones((16,), jnp.int32)
        pltpu.sync_copy(epk_ref.at[pl.ds(sid * CPT, CPT)], pk_v)
        plsc.subcore_barrier()

        def step(t, carry):
            for r in range(2):
                for k in range(CHUNK // 16):
                    w = pk_v[t * 2 + r, pl.ds(k * 16, 16)]
                    uidx_v[pl.ds(r * CHUNK + k * 16, 16)] = (
                        w >> 16 if high else w & 0xFFFF)
            pltpu.sync_copy(ones_v, hist_s.at[uidx_v], add=True)
            return carry

        lax.fori_loop(0, CPT // 2, step, 0)
        plsc.subcore_barrier()
        pltpu.sync_copy(hist_s.at[pl.ds(sid * RPT, RPT)], row_v)
        pltpu.sync_copy(row_v, out_ref.at[pl.ds(sid * RPT, RPT)])

    @pl.when(cid == 0)
    def _():
        phase(True, cr_ref)

    @pl.when(cid == 1)
    def _():
        phase(False, cc_ref)


@functools.partial(
    pl.kernel,
    out_type=tuple(jax.ShapeDtypeStruct((NPAD, 128), jnp.float32)
                   for _ in range(4)),
    mesh=_MESH,
    scratch_types=[
        pltpu.VMEM((CPT, CHUNK), jnp.int32),
        pltpu.VMEM((NBUF, CHUNK), jnp.int32),
        pltpu.VMEM((NBUF, CHUNK), jnp.int32),
        pltpu.VMEM((NBUF, CHUNK, 128), jnp.float32),
        pltpu.VMEM_SHARED((NPAD, 128), jnp.float32),
        pltpu.SemaphoreType.DMA,
        pltpu.SemaphoreType.DMA,
        pltpu.SemaphoreType.DMA,
        pltpu.SemaphoreType.DMA,
        pltpu.SemaphoreType.DMA,
        pltpu.SemaphoreType.DMA,
    ],
)
def _sc_scatter(epk_ref, mil_ref, mir_ref, mol_ref, mor_ref,
                pil_ref, pir_ref, pol_ref, por_ref,
                pk_v, sidx_v, didx_v, g_v, acc_s,
                sem0, sem1, sem2, sem3, sem4, sem5):
    cid = lax.axis_index("c")
    sid = lax.axis_index("s")
    sems = (sem0, sem1, sem2)
    ssems = (sem3, sem4, sem5)

    NRC = RPT // CHUNK

    def do_phase(m_ref, p_ref, src_high):
        for q in range(NBUF):
            off = sid * RPT + q * CHUNK
            pltpu.async_copy(m_ref.at[pl.ds(off, CHUNK)], g_v.at[q], sems[q])
        for k in range(NRC):
            b = k % NBUF
            off = sid * RPT + k * CHUNK
            pltpu.make_async_copy(
                m_ref.at[pl.ds(off, CHUNK)], g_v.at[b], sems[b]).wait()
            pltpu.sync_copy(g_v.at[b], acc_s.at[pl.ds(off, CHUNK)])
            if k + NBUF < NRC:
                noff = sid * RPT + (k + NBUF) * CHUNK
                pltpu.async_copy(
                    m_ref.at[pl.ds(noff, CHUNK)], g_v.at[b], sems[b])
        plsc.subcore_barrier()

        def unpack_fire(j, b, first):
            if not first:
                pltpu.make_async_copy(
                    g_v.at[b], acc_s.at[didx_v.at[b]], ssems[b]).wait()
            for k in range(CHUNK // 16):
                w = pk_v[j, pl.ds(k * 16, 16)]
                sidx_v[b, pl.ds(k * 16, 16)] = (
                    w >> 16 if src_high else w & 0xFFFF)
                didx_v[b, pl.ds(k * 16, 16)] = (
                    w & 0xFFFF if src_high else w >> 16)
            pltpu.async_copy(m_ref.at[sidx_v.at[b]], g_v.at[b], sems[b])

        for b in range(NBUF):
            unpack_fire(b, b, True)

        def step(t, carry):
            for b in range(NBUF):
                j = t * NBUF + b
                pltpu.make_async_copy(
                    m_ref.at[sidx_v.at[b]], g_v.at[b], sems[b]).wait()
                pltpu.async_copy(
                    g_v.at[b], acc_s.at[didx_v.at[b]], ssems[b], add=True)

                @pl.when(j + NBUF < CPT)
                def _():
                    unpack_fire(j + NBUF, b, False)
            return carry

        lax.fori_loop(0, NFT, step, 0)
        for r in range(REM):
            jr = NFT * NBUF + r
            br = jr % NBUF
            pltpu.make_async_copy(
                m_ref.at[sidx_v.at[br]], g_v.at[br], sems[br]).wait()
            pltpu.async_copy(
                g_v.at[br], acc_s.at[didx_v.at[br]], ssems[br], add=True)
        for b in range(NBUF):
            pltpu.make_async_copy(
                g_v.at[b], acc_s.at[didx_v.at[b]], ssems[b]).wait()
        plsc.subcore_barrier()
        for k in range(NRC):
            b = k % NBUF
            off = sid * RPT + k * CHUNK
            if k >= NBUF:
                poff = sid * RPT + (k - NBUF) * CHUNK
                pltpu.make_async_copy(
                    g_v.at[b], p_ref.at[pl.ds(poff, CHUNK)], sems[b]).wait()
            pltpu.sync_copy(acc_s.at[pl.ds(off, CHUNK)], g_v.at[b])
            pltpu.async_copy(g_v.at[b], p_ref.at[pl.ds(off, CHUNK)], sems[b])
        for k in range(NRC - NBUF, NRC):
            b = k % NBUF
            off = sid * RPT + k * CHUNK
            pltpu.make_async_copy(
                g_v.at[b], p_ref.at[pl.ds(off, CHUNK)], sems[b]).wait()
        plsc.subcore_barrier()

    pltpu.sync_copy(epk_ref.at[pl.ds(sid * CPT, CPT)], pk_v)

    @pl.when(cid == 0)
    def _():
        do_phase(mil_ref, pil_ref, True)
        do_phase(mol_ref, pol_ref, False)

    @pl.when(cid == 1)
    def _():
        do_phase(mir_ref, pir_ref, True)
        do_phase(mor_ref, por_ref, False)


def _tc_prep_body(x_ref, w_ref, dc_ref, dr_ref, mil, mir, mol, mor, di, do_):
    h = jnp.dot(x_ref[...], w_ref[...], preferred_element_type=jnp.float32)
    din = lax.rsqrt(dc_ref[...].astype(jnp.float32) + 1.0)
    dou = lax.rsqrt(dr_ref[...].astype(jnp.float32) + 1.0)
    mil[...] = h[:, 0:128] * din
    mir[...] = h[:, 128:256] * din
    mol[...] = h[:, 256:384] * dou
    mor[...] = h[:, 384:512] * dou
    di[...] = din
    do_[...] = dou


_tc_prep = pl.pallas_call(
    _tc_prep_body,
    grid=(NPAD // BR,),
    in_specs=[
        pl.BlockSpec((BR, D), lambda i: (i, 0)),

        pl.BlockSpec((D, 2 * D), lambda i: (0, 0)),
        pl.BlockSpec((BR, 1), lambda i: (i, 0)),
        pl.BlockSpec((BR, 1), lambda i: (i, 0)),
    ],
    out_specs=[
        pl.BlockSpec((BR, 128), lambda i: (i, 0)),
        pl.BlockSpec((BR, 128), lambda i: (i, 0)),
        pl.BlockSpec((BR, 128), lambda i: (i, 0)),
        pl.BlockSpec((BR, 128), lambda i: (i, 0)),
        pl.BlockSpec((BR, 1), lambda i: (i, 0)),
        pl.BlockSpec((BR, 1), lambda i: (i, 0)),
    ],
    out_shape=[jax.ShapeDtypeStruct((NPAD, 128), jnp.float32)] * 4
    + [jax.ShapeDtypeStruct((NPAD, 1), jnp.float32)] * 2,
)


def _tc_final_body(pil, pir, pol, por, di, do_, wfc, bi, bo, bf, out):
    pin = jnp.concatenate([pil[...], pir[...]], axis=1) * di[...]
    pou = jnp.concatenate([pol[...], por[...]], axis=1) * do_[...]
    big = jnp.concatenate([pin, pou], axis=1)
    y = jnp.dot(big, wfc[...], preferred_element_type=jnp.float32)
    bias = (jnp.dot(bi[...], wfc[0:D, :], preferred_element_type=jnp.float32)
            + jnp.dot(bo[...], wfc[D:2 * D, :],
                      preferred_element_type=jnp.float32)
            + bf[...])
    out[...] = jnp.maximum(y + bias, 0.0)


_tc_final = pl.pallas_call(
    _tc_final_body,
    grid=(NPAD // BR,),
    in_specs=[
        pl.BlockSpec((BR, 128), lambda i: (i, 0)),
        pl.BlockSpec((BR, 128), lambda i: (i, 0)),
        pl.BlockSpec((BR, 128), lambda i: (i, 0)),
        pl.BlockSpec((BR, 128), lambda i: (i, 0)),
        pl.BlockSpec((BR, 1), lambda i: (i, 0)),
        pl.BlockSpec((BR, 1), lambda i: (i, 0)),
        pl.BlockSpec((2 * D, D), lambda i: (0, 0)),
        pl.BlockSpec((1, D), lambda i: (0, 0)),
        pl.BlockSpec((1, D), lambda i: (0, 0)),
        pl.BlockSpec((1, D), lambda i: (0, 0)),
    ],
    out_specs=pl.BlockSpec((BR, D), lambda i: (i, 0)),
    out_shape=jax.ShapeDtypeStruct((N, D), jnp.float32),
)


def kernel(x, edge_index, W_in, b_in, W_out, b_out, W_fc, b_fc):
    ei = edge_index.astype(jnp.int32)
    pad_idx = N + (jnp.arange(EPAD - E, dtype=jnp.int32) % (NPAD - N))
    erow = jnp.concatenate([ei[0], pad_idx])
    ecol = jnp.concatenate([ei[1], pad_idx])
    epk = ((erow << 16) | ecol).reshape(NCH, CHUNK)
    Wcat = jnp.concatenate([W_in, W_out], axis=1)
    cnt_row, cnt_col = _sc_degrees(epk)
    mil, mir, mol, mor, din, dou = _tc_prep(
        x, Wcat, cnt_col[:, None], cnt_row[:, None])
    pil, pir, pol, por = _sc_scatter(epk, mil, mir, mol, mor)
    return _tc_final(pil, pir, pol, por, din, dou, W_fc,
                     b_in[None, :], b_out[None, :], b_fc[None, :])
